# baseline (speedup 1.0000x reference)
import functools

import jax
import jax.numpy as jnp
from jax import lax
from jax.experimental import pallas as pl
from jax.experimental.pallas import tpu as pltpu
from jax.experimental.pallas import tpu_sc as plsc

BATCH = 16384
HIST_LEN = 50
EMBED_DIM = 32

B = BATCH * HIST_LEN
SUB = 128
IDX_ROWS = B // SUB
NC, NS = 2, 16
NW = NC * NS
ROWS_PER_W = IDX_ROWS // NW

_MESH = plsc.VectorSubcoreMesh(core_axis_name="c", subcore_axis_name="s")


@functools.partial(
    pl.kernel,
    mesh=_MESH,
    out_type=jax.ShapeDtypeStruct((B, EMBED_DIM), jnp.float32),
    compiler_params=pltpu.CompilerParams(use_tc_tiling_on_sc=False),
    scratch_types=[
        pltpu.VMEM((SUB,), jnp.int32),
        pltpu.VMEM((SUB, EMBED_DIM), jnp.float32),
        pltpu.SemaphoreType.DMA,
    ],
)
def _gather_sc(table_hbm, idx_hbm, out_hbm, idx_v, rows_v, sem):
    wid = lax.axis_index("s") * NC + lax.axis_index("c")
    row0 = wid * ROWS_PER_W

    def step(i, carry):
        r = row0 + i
        pltpu.sync_copy(idx_hbm.at[r], idx_v)
        pltpu.async_copy(table_hbm.at[idx_v], rows_v, sem).wait()
        pltpu.sync_copy(rows_v, out_hbm.at[pl.ds(r * SUB, SUB)])
        return carry

    lax.fori_loop(0, ROWS_PER_W, step, 0)


def kernel(nouns_idx_tensor, histwords_embeddings):
    idx = nouns_idx_tensor.reshape(-1).astype(jnp.int32).reshape(IDX_ROWS, SUB)
    out = _gather_sc(histwords_embeddings, idx)
    return out.reshape(BATCH, HIST_LEN, EMBED_DIM)


# speedup vs baseline: 1.3369x; 1.3369x over previous
import functools

import jax
import jax.numpy as jnp
from jax import lax
from jax.experimental import pallas as pl
from jax.experimental.pallas import tpu as pltpu
from jax.experimental.pallas import tpu_sc as plsc

BATCH = 16384
HIST_LEN = 50
EMBED_DIM = 32

B = BATCH * HIST_LEN
SUB = 128
IDX_ROWS = B // SUB
NC, NS = 2, 16
NW = NC * NS
ROWS_PER_W = IDX_ROWS // NW
K = 8
NSUP = ROWS_PER_W // K

_MESH = plsc.VectorSubcoreMesh(core_axis_name="c", subcore_axis_name="s")


@functools.partial(
    pl.kernel,
    mesh=_MESH,
    out_type=jax.ShapeDtypeStruct((IDX_ROWS, SUB, EMBED_DIM), jnp.float32),
    compiler_params=pltpu.CompilerParams(use_tc_tiling_on_sc=False),
    scratch_types=[
        pltpu.VMEM((2, K, SUB), jnp.int32),
        pltpu.VMEM((2, K, SUB, EMBED_DIM), jnp.float32),
        pltpu.SemaphoreType.DMA((2,)),
        pltpu.SemaphoreType.DMA((2,)),
        pltpu.SemaphoreType.DMA((2,)),
    ],
)
def _gather_sc(table_hbm, idx_hbm, out_hbm, idx_v, rows_v, sem_i, sem_g, sem_o):
    wid = lax.axis_index("s") * NC + lax.axis_index("c")
    row0 = wid * ROWS_PER_W

    def idx_copy(g, p):
        return pltpu.make_async_copy(
            idx_hbm.at[pl.ds(row0 + g * K, K)], idx_v.at[p], sem_i.at[p])

    def out_copy(g, p):
        return pltpu.make_async_copy(
            rows_v.at[p], out_hbm.at[pl.ds(row0 + g * K, K)], sem_o.at[p])

    def do_super(g, p):
        gathers = [
            pltpu.make_async_copy(
                table_hbm.at[idx_v.at[p, j]], rows_v.at[p, j], sem_g.at[p])
            for j in range(K)
        ]
        for c in gathers:
            c.start()
        for c in gathers:
            c.wait()
        out_copy(g, p).start()

    idx_copy(0, 0).start()
    idx_copy(1, 1).start()
    idx_copy(0, 0).wait()
    do_super(0, 0)
    idx_copy(2, 0).start()
    idx_copy(1, 1).wait()
    do_super(1, 1)

    def step(g, carry):
        p = lax.rem(g, 2)
        idx_copy(g + 1, 1 - p).start()
        out_copy(g - 2, p).wait()
        idx_copy(g, p).wait()
        do_super(g, p)
        return carry

    lax.fori_loop(2, NSUP - 1, step, 0)

    pL = (NSUP - 1) % 2
    out_copy(NSUP - 3, pL).wait()
    idx_copy(NSUP - 1, pL).wait()
    do_super(NSUP - 1, pL)
    out_copy(NSUP - 2, (NSUP - 2) % 2).wait()
    out_copy(NSUP - 1, pL).wait()


def kernel(nouns_idx_tensor, histwords_embeddings):
    idx = nouns_idx_tensor.reshape(-1).astype(jnp.int32).reshape(IDX_ROWS, SUB)
    out = _gather_sc(histwords_embeddings, idx)
    return out.reshape(BATCH, HIST_LEN, EMBED_DIM)


# speedup vs baseline: 1.6873x; 1.2621x over previous
import functools

import jax
import jax.numpy as jnp
from jax import lax
from jax.experimental import pallas as pl
from jax.experimental.pallas import tpu as pltpu
from jax.experimental.pallas import tpu_sc as plsc

BATCH = 16384
HIST_LEN = 50
EMBED_DIM = 32

SUB = 128
SG = BATCH // SUB
UNITS = HIST_LEN * SG
NC, NS = 2, 16
NW = NC * NS
UPW = UNITS // NW
DG = EMBED_DIM // 8

_MESH = plsc.VectorSubcoreMesh(core_axis_name="c", subcore_axis_name="s")


@functools.partial(
    pl.kernel,
    mesh=_MESH,
    out_type=jax.ShapeDtypeStruct((HIST_LEN, DG, SG, 8, SUB), jnp.float32),
    compiler_params=pltpu.CompilerParams(
        use_tc_tiling_on_sc=False, needs_layout_passes=False),
    scratch_types=[
        pltpu.VMEM((UPW, SUB), jnp.int32),
        pltpu.VMEM((4, SUB, EMBED_DIM), jnp.float32),
        pltpu.VMEM((2, DG, 8, SUB), jnp.float32),
        pltpu.SemaphoreType.DMA((4,)),
        pltpu.SemaphoreType.DMA((2,)),
    ],
)
def _gather_sc(table_hbm, idx_hbm, out_hbm, idx_all, rows_g, rows_t, sem_g, sem_o):
    wid = lax.axis_index("s") * NC + lax.axis_index("c")
    u0 = wid * UPW

    pltpu.sync_copy(idx_hbm.at[pl.ds(u0, UPW)], idx_all)

    def gather(t):
        q = lax.rem(t, 4)
        return pltpu.make_async_copy(
            table_hbm.at[idx_all.at[t]], rows_g.at[q], sem_g.at[q])

    def store(t, dg):
        u = u0 + t
        h = lax.div(u, SG)
        sg = lax.rem(u, SG)
        q = lax.rem(t, 2)
        return pltpu.make_async_copy(
            rows_t.at[q, dg], out_hbm.at[h, dg, sg], sem_o.at[q])

    riota = [lax.iota(jnp.int32, 16) + 16 * k for k in range(8)]

    gather(0).start()
    gather(1).start()
    gather(2).start()

    def unit(t, carry):
        q4 = lax.rem(t, 4)
        q2 = lax.rem(t, 2)

        @pl.when(t >= 2)
        def _():
            for dg in range(DG):
                store(t - 2, dg).wait()

        gather(t).wait()

        @pl.when(t + 3 < UPW)
        def _():
            gather(t + 3).start()

        src = rows_g.at[q4]

        def col(d, c):
            cvec = jnp.full((16,), 0, jnp.int32) + d
            dg = lax.div(d, 8)
            dr = lax.rem(d, 8)
            for k in range(8):
                v = plsc.load_gather(src, [riota[k], cvec])
                rows_t[q2, dg, dr, pl.ds(16 * k, 16)] = v
            return c

        lax.fori_loop(0, EMBED_DIM, col, 0)

        for dg in range(DG):
            store(t, dg).start()
        return carry

    lax.fori_loop(0, UPW, unit, 0)

    for dg in range(DG):
        store(UPW - 2, dg).wait()
        store(UPW - 1, dg).wait()


def kernel(nouns_idx_tensor, histwords_embeddings):
    idx = nouns_idx_tensor.astype(jnp.int32).T.reshape(UNITS, SUB)
    out5 = _gather_sc(histwords_embeddings, idx)
    return out5.transpose(2, 4, 0, 1, 3).reshape(BATCH, HIST_LEN, EMBED_DIM)
